# baseline (speedup 1.0000x reference)
import functools

import jax
import jax.numpy as jnp
from jax import lax
from jax.experimental import pallas as pl
from jax.experimental.pallas import tpu as pltpu
from jax.experimental.pallas import tpu_sc as plsc

N = 10000
D = 128
L = 8
NPAD = 10240
NC, NS = 2, 16
NW = NC * NS
CH = 64
EPW = 10432
EPAD = EPW * NW
RPT = NPAD // NS
BN = 320
GRID = NPAD // BN
SLOPE = 0.2


def _leaky(v):
    return jnp.where(v >= 0, v, SLOPE * v)


_LOG2E = 1.4426950408889634
_E2C = [1.0, 0.6931471805599453, 0.2402265069591007, 0.05550410866482158,
        0.009618129107628477, 0.001333355814642844, 0.00015403530393381608]


def _exp_neg(d):
    z = jnp.maximum(d * _LOG2E, -126.0)
    n = (z - 0.5).astype(jnp.int32)
    f = z - n.astype(jnp.float32)
    p = jnp.full((16,), _E2C[6], jnp.float32)
    for c in (_E2C[5], _E2C[4], _E2C[3], _E2C[2], _E2C[1], _E2C[0]):
        p = p * f + c
    scale = jax.lax.bitcast_convert_type(
        jax.lax.shift_left(n + 127, 23), jnp.float32)
    return p * scale


def _acc_gmax(gmax_ref, i, es):
    @pl.when(i == 0)
    def _():
        gmax_ref[...] = jnp.full((1, 1), -jnp.inf, jnp.float32)

    gmax_ref[...] = jnp.maximum(gmax_ref[...], jnp.max(es, keepdims=True))


def _tc_first_body(x_ref, wp_ref, bp_ref, w_ref, asrc_ref, adst_ref,
                   g_ref, es_ref, ed_ref, gmax_ref):
    i = pl.program_id(0)
    h = jnp.dot(x_ref[...], wp_ref[...], preferred_element_type=jnp.float32, precision=jax.lax.Precision.HIGHEST)
    h = h + bp_ref[...]
    g = jnp.dot(h, w_ref[...], preferred_element_type=jnp.float32, precision=jax.lax.Precision.HIGHEST)
    g_ref[...] = g
    es = jnp.dot(g, asrc_ref[...], preferred_element_type=jnp.float32, precision=jax.lax.Precision.HIGHEST)
    ed = jnp.dot(g, adst_ref[...], preferred_element_type=jnp.float32, precision=jax.lax.Precision.HIGHEST)
    es_ref[...] = es
    ed_ref[...] = ed
    _acc_gmax(gmax_ref, i, es)


_tc_first = pl.pallas_call(
    _tc_first_body,
    grid=(GRID,),
    in_specs=[
        pl.BlockSpec((BN, D), lambda i: (i, 0)),
        pl.BlockSpec((D, D), lambda i: (0, 0)),
        pl.BlockSpec((1, D), lambda i: (0, 0)),
        pl.BlockSpec((D, D), lambda i: (0, 0)),
        pl.BlockSpec((D, 1), lambda i: (0, 0)),
        pl.BlockSpec((D, 1), lambda i: (0, 0)),
    ],
    out_specs=[
        pl.BlockSpec((BN, D), lambda i: (i, 0)),
        pl.BlockSpec((BN, 1), lambda i: (i, 0)),
        pl.BlockSpec((BN, 1), lambda i: (i, 0)),
        pl.BlockSpec((1, 1), lambda i: (0, 0)),
    ],
    out_shape=[
        jax.ShapeDtypeStruct((NPAD, D), jnp.float32),
        jax.ShapeDtypeStruct((NPAD, 1), jnp.float32),
        jax.ShapeDtypeStruct((NPAD, 1), jnp.float32),
        jax.ShapeDtypeStruct((1, 1), jnp.float32),
    ],
)


def _tc_mid_body(a0_ref, a1_ref, s0_ref, s1_ref, b_ref, w_ref, asrc_ref,
                 adst_ref, g_ref, es_ref, ed_ref, gmax_ref):
    i = pl.program_id(0)
    s = s0_ref[0, 0] + s1_ref[0, 0]
    acc = a0_ref[...] + a1_ref[...]
    h = acc / (s + 1e-16)[:, None] + b_ref[...]
    h = jnp.maximum(h, 0.0)
    rows = i * BN + lax.broadcasted_iota(jnp.int32, (BN, 1), 0)
    h = jnp.where(rows < N, h, 0.0)
    g = jnp.dot(h, w_ref[...], preferred_element_type=jnp.float32, precision=jax.lax.Precision.HIGHEST)
    g_ref[...] = g
    es = jnp.dot(g, asrc_ref[...], preferred_element_type=jnp.float32, precision=jax.lax.Precision.HIGHEST)
    ed = jnp.dot(g, adst_ref[...], preferred_element_type=jnp.float32, precision=jax.lax.Precision.HIGHEST)
    es_ref[...] = es
    ed_ref[...] = ed
    _acc_gmax(gmax_ref, i, es)


_tc_mid = pl.pallas_call(
    _tc_mid_body,
    grid=(GRID,),
    in_specs=[
        pl.BlockSpec((BN, D), lambda i: (i, 0)),
        pl.BlockSpec((BN, D), lambda i: (GRID + i, 0)),
        pl.BlockSpec((1, 1, BN), lambda i: (i, 0, 0)),
        pl.BlockSpec((1, 1, BN), lambda i: (GRID + i, 0, 0)),
        pl.BlockSpec((1, D), lambda i: (0, 0)),
        pl.BlockSpec((D, D), lambda i: (0, 0)),
        pl.BlockSpec((D, 1), lambda i: (0, 0)),
        pl.BlockSpec((D, 1), lambda i: (0, 0)),
    ],
    out_specs=[
        pl.BlockSpec((BN, D), lambda i: (i, 0)),
        pl.BlockSpec((BN, 1), lambda i: (i, 0)),
        pl.BlockSpec((BN, 1), lambda i: (i, 0)),
        pl.BlockSpec((1, 1), lambda i: (0, 0)),
    ],
    out_shape=[
        jax.ShapeDtypeStruct((NPAD, D), jnp.float32),
        jax.ShapeDtypeStruct((NPAD, 1), jnp.float32),
        jax.ShapeDtypeStruct((NPAD, 1), jnp.float32),
        jax.ShapeDtypeStruct((1, 1), jnp.float32),
    ],
)


def _tc_final_body(a0_ref, a1_ref, s0_ref, s1_ref, b_ref, out_ref):
    s = s0_ref[0, 0] + s1_ref[0, 0]
    acc = a0_ref[...] + a1_ref[...]
    h = acc / (s + 1e-16)[:, None] + b_ref[...]
    nrm = jnp.sqrt(jnp.sum(h * h, axis=1, keepdims=True))
    out_ref[...] = h / jnp.maximum(nrm, 1e-12)


_tc_final = pl.pallas_call(
    _tc_final_body,
    grid=(GRID,),
    in_specs=[
        pl.BlockSpec((BN, D), lambda i: (i, 0)),
        pl.BlockSpec((BN, D), lambda i: (GRID + i, 0)),
        pl.BlockSpec((1, 1, BN), lambda i: (i, 0, 0)),
        pl.BlockSpec((1, 1, BN), lambda i: (GRID + i, 0, 0)),
        pl.BlockSpec((1, D), lambda i: (0, 0)),
    ],
    out_specs=pl.BlockSpec((BN, D), lambda i: (i, 0)),
    out_shape=jax.ShapeDtypeStruct((NPAD, D), jnp.float32),
)


_sc_mesh = plsc.VectorSubcoreMesh(
    core_axis_name="c", subcore_axis_name="s", num_cores=NC, num_subcores=NS)


@functools.partial(
    pl.kernel,
    mesh=_sc_mesh,
    out_type=[
        jax.ShapeDtypeStruct((NC * NPAD, D), jnp.float32),
        jax.ShapeDtypeStruct((NC * NPAD,), jnp.float32),
    ],
    scratch_types=[
        pltpu.VMEM((NPAD,), jnp.float32),
        pltpu.VMEM((NPAD,), jnp.float32),
        pltpu.VMEM((16,), jnp.float32),
        [pltpu.VMEM((CH,), jnp.int32) for _ in range(3)],
        [pltpu.VMEM((CH,), jnp.int32) for _ in range(3)],
        [pltpu.VMEM((CH + 16,), jnp.float32) for _ in range(3)],
        [pltpu.VMEM((CH, D), jnp.float32) for _ in range(3)],
        pltpu.VMEM_SHARED((NPAD, D), jnp.float32),
        pltpu.VMEM_SHARED((NPAD,), jnp.float32),
        [pltpu.SemaphoreType.DMA for _ in range(3)],
        [pltpu.SemaphoreType.DMA for _ in range(3)],
        [pltpu.SemaphoreType.DMA for _ in range(3)],
    ],
    compiler_params=pltpu.CompilerParams(needs_layout_passes=False),
)
def _sc_edge(src_hbm, dst_hbm, g_hbm, es_hbm, ed_hbm, gmax_hbm, zr_hbm, z1_hbm,
             acc_out, s_out,
             es_v, ed_v, gmax_v, idx_ss, idx_ds, w_vs, rows_vs,
             acc_sh, s_sh, semGs, semSs, semWs):
    cid = lax.axis_index("c")
    sid = lax.axis_index("s")
    r0 = sid * RPT
    pltpu.sync_copy(zr_hbm.at[pl.ds(r0, RPT)], acc_sh.at[pl.ds(r0, RPT)])
    pltpu.sync_copy(z1_hbm.at[pl.ds(r0, RPT)], s_sh.at[pl.ds(r0, RPT)])
    pltpu.sync_copy(es_hbm, es_v)
    pltpu.sync_copy(ed_hbm, ed_v)
    pltpu.sync_copy(gmax_hbm, gmax_v)
    plsc.subcore_barrier()

    gmax = gmax_v[...]
    base = (cid * NS + sid) * EPW
    nch = EPW // CH
    bufs = tuple(
        (idx_ss[i], idx_ds[i], w_vs[i], rows_vs[i], semGs[i], semSs[i],
         semWs[i]) for i in range(3))

    def fetch(j, b):
        idx_s, idx_d, _, rows_v, semG, _, _ = b
        off = base + j * CH
        pltpu.sync_copy(src_hbm.at[pl.ds(off, CH)], idx_s)
        pltpu.sync_copy(dst_hbm.at[pl.ds(off, CH)], idx_d)
        pltpu.async_copy(g_hbm.at[idx_s], rows_v, semG)

    def wait_scatter(b):
        _, idx_d, w_v, rows_v, _, semS, semW = b
        pltpu.make_async_copy(rows_v, acc_sh.at[idx_d], semS).wait()
        pltpu.make_async_copy(w_v.at[pl.ds(16, CH)], s_sh.at[idx_d],
                              semW).wait()

    def proc(u, b, b2):
        idx_s, idx_d, w_v, rows_v, semG, semS, semW = b
        pltpu.make_async_copy(g_hbm.at[idx_s], rows_v, semG).wait()
        for k in range(CH // 16):
            sv = idx_s[pl.ds(k * 16, 16)]
            dv = idx_d[pl.ds(k * 16, 16)]
            es16 = plsc.load_gather(es_v, [sv])
            ed16 = plsc.load_gather(ed_v, [dv])
            e = _leaky(es16 + ed16)
            c = _leaky(gmax + ed16)
            w_v[pl.ds(16 + k * 16, 16)] = _exp_neg(e - c)
        for r in range(CH):
            wr = plsc.load_gather(
                w_v, [jnp.full((16,), 16 + r, dtype=jnp.int32)])
            for q in range(D // 16):
                rows_v[r, pl.ds(q * 16, 16)] = rows_v[r, pl.ds(q * 16, 16)] * wr

        @pl.when(u >= 1)
        def _():
            wait_scatter(b2)

        @pl.when(u + 2 < nch)
        def _():
            fetch(u + 2, b2)

        pltpu.async_copy(rows_v, acc_sh.at[idx_d], semS, add=True)
        pltpu.async_copy(w_v.at[pl.ds(16, CH)], s_sh.at[idx_d], semW,
                         add=True)

    fetch(0, bufs[0])
    fetch(1, bufs[1])

    def tri(t, carry):
        u = 3 * t
        proc(u, bufs[0], bufs[2])
        proc(u + 1, bufs[1], bufs[0])
        proc(u + 2, bufs[2], bufs[1])
        return carry

    lax.fori_loop(0, (nch - 1) // 3, tri, 0)
    proc(nch - 1, bufs[0], bufs[2])
    wait_scatter(bufs[0])
    plsc.subcore_barrier()
    out_r0 = cid * NPAD + r0
    pltpu.sync_copy(acc_sh.at[pl.ds(r0, RPT)], acc_out.at[pl.ds(out_r0, RPT)])
    pltpu.sync_copy(s_sh.at[pl.ds(r0, RPT)], s_out.at[pl.ds(out_r0, RPT)])


def kernel(x, edge_index, Wp, bp, Ws, a_src, a_dst, bs):
    xp = jnp.concatenate([x, jnp.zeros((NPAD - N, D), jnp.float32)], axis=0)
    loops = jnp.arange(N, dtype=jnp.int32)
    npad_e = EPAD - (edge_index.shape[1] + N)
    src = jnp.concatenate([
        edge_index[0], loops, jnp.zeros((npad_e,), jnp.int32)])
    dst = jnp.concatenate([
        edge_index[1], loops,
        N + (jnp.arange(npad_e, dtype=jnp.int32) % (NPAD - N))])
    zr = jnp.zeros((NPAD, D), jnp.float32)
    z1 = jnp.zeros((NPAD,), jnp.float32)

    g, es, ed, gmax = _tc_first(
        xp, Wp, bp[None, :], Ws[0], a_src[0][:, None], a_dst[0][:, None])
    for i in range(L):
        gmax16 = jnp.broadcast_to(gmax.reshape(1), (16,))
        acc, s = _sc_edge(src, dst, g, es.reshape(NPAD), ed.reshape(NPAD),
                          gmax16, zr, z1)
        s3 = s.reshape(NC * GRID, 1, BN)
        if i < L - 1:
            g, es, ed, gmax = _tc_mid(
                acc, acc, s3, s3, bs[i][None, :], Ws[i + 1],
                a_src[i + 1][:, None], a_dst[i + 1][:, None])
        else:
            out = _tc_final(acc, acc, s3, s3, bs[i][None, :])
    return out[:N]


# speedup vs baseline: 1.0045x; 1.0045x over previous
import functools

import jax
import jax.numpy as jnp
from jax import lax
from jax.experimental import pallas as pl
from jax.experimental.pallas import tpu as pltpu
from jax.experimental.pallas import tpu_sc as plsc

N = 10000
D = 128
L = 8
NPAD = 10240
NC, NS = 2, 16
NW = NC * NS
CH = 64
EPW = 10432
EPAD = EPW * NW
RPT = NPAD // NS
BN = 320
GRID = NPAD // BN
SLOPE = 0.2


def _leaky(v):
    return jnp.where(v >= 0, v, SLOPE * v)


_LOG2E = 1.4426950408889634
_E2C = [1.0, 0.6931471805599453, 0.2402265069591007, 0.05550410866482158,
        0.009618129107628477, 0.001333355814642844, 0.00015403530393381608]


def _exp_neg(d):
    z = jnp.maximum(d * _LOG2E, -126.0)
    n = (z - 0.5).astype(jnp.int32)
    f = z - n.astype(jnp.float32)
    p = jnp.full((16,), _E2C[6], jnp.float32)
    for c in (_E2C[5], _E2C[4], _E2C[3], _E2C[2], _E2C[1], _E2C[0]):
        p = p * f + c
    scale = jax.lax.bitcast_convert_type(
        jax.lax.shift_left(n + 127, 23), jnp.float32)
    return p * scale


def _acc_gmax(gmax_ref, i, es):
    @pl.when(i == 0)
    def _():
        gmax_ref[...] = jnp.full((1, 1), -jnp.inf, jnp.float32)

    gmax_ref[...] = jnp.maximum(gmax_ref[...], jnp.max(es, keepdims=True))


def _tc_first_body(x_ref, wp_ref, bp_ref, w_ref, asrc_ref, adst_ref,
                   g_ref, es_ref, ed_ref, gmax_ref):
    i = pl.program_id(0)
    h = jnp.dot(x_ref[...], wp_ref[...], preferred_element_type=jnp.float32, precision=jax.lax.Precision.HIGHEST)
    h = h + bp_ref[...]
    g = jnp.dot(h, w_ref[...], preferred_element_type=jnp.float32, precision=jax.lax.Precision.HIGHEST)
    g_ref[...] = g
    es = jnp.dot(g, asrc_ref[...], preferred_element_type=jnp.float32, precision=jax.lax.Precision.HIGHEST)
    ed = jnp.dot(g, adst_ref[...], preferred_element_type=jnp.float32, precision=jax.lax.Precision.HIGHEST)
    es_ref[...] = es
    ed_ref[...] = ed
    _acc_gmax(gmax_ref, i, es)


_tc_first = pl.pallas_call(
    _tc_first_body,
    grid=(GRID,),
    in_specs=[
        pl.BlockSpec((BN, D), lambda i: (i, 0)),
        pl.BlockSpec((D, D), lambda i: (0, 0)),
        pl.BlockSpec((1, D), lambda i: (0, 0)),
        pl.BlockSpec((D, D), lambda i: (0, 0)),
        pl.BlockSpec((D, 1), lambda i: (0, 0)),
        pl.BlockSpec((D, 1), lambda i: (0, 0)),
    ],
    out_specs=[
        pl.BlockSpec((BN, D), lambda i: (i, 0)),
        pl.BlockSpec((BN, 1), lambda i: (i, 0)),
        pl.BlockSpec((BN, 1), lambda i: (i, 0)),
        pl.BlockSpec((1, 1), lambda i: (0, 0)),
    ],
    out_shape=[
        jax.ShapeDtypeStruct((NPAD, D), jnp.float32),
        jax.ShapeDtypeStruct((NPAD, 1), jnp.float32),
        jax.ShapeDtypeStruct((NPAD, 1), jnp.float32),
        jax.ShapeDtypeStruct((1, 1), jnp.float32),
    ],
)


def _tc_mid_body(a0_ref, a1_ref, s0_ref, s1_ref, b_ref, w_ref, asrc_ref,
                 adst_ref, g_ref, es_ref, ed_ref, gmax_ref):
    i = pl.program_id(0)
    s = s0_ref[0, 0] + s1_ref[0, 0]
    acc = a0_ref[...] + a1_ref[...]
    h = acc / (s + 1e-16)[:, None] + b_ref[...]
    h = jnp.maximum(h, 0.0)
    rows = i * BN + lax.broadcasted_iota(jnp.int32, (BN, 1), 0)
    h = jnp.where(rows < N, h, 0.0)
    g = jnp.dot(h, w_ref[...], preferred_element_type=jnp.float32, precision=jax.lax.Precision.HIGHEST)
    g_ref[...] = g
    es = jnp.dot(g, asrc_ref[...], preferred_element_type=jnp.float32, precision=jax.lax.Precision.HIGHEST)
    ed = jnp.dot(g, adst_ref[...], preferred_element_type=jnp.float32, precision=jax.lax.Precision.HIGHEST)
    es_ref[...] = es
    ed_ref[...] = ed
    _acc_gmax(gmax_ref, i, es)


_tc_mid = pl.pallas_call(
    _tc_mid_body,
    grid=(GRID,),
    in_specs=[
        pl.BlockSpec((BN, D), lambda i: (i, 0)),
        pl.BlockSpec((BN, D), lambda i: (GRID + i, 0)),
        pl.BlockSpec((1, 1, BN), lambda i: (i, 0, 0)),
        pl.BlockSpec((1, 1, BN), lambda i: (GRID + i, 0, 0)),
        pl.BlockSpec((1, D), lambda i: (0, 0)),
        pl.BlockSpec((D, D), lambda i: (0, 0)),
        pl.BlockSpec((D, 1), lambda i: (0, 0)),
        pl.BlockSpec((D, 1), lambda i: (0, 0)),
    ],
    out_specs=[
        pl.BlockSpec((BN, D), lambda i: (i, 0)),
        pl.BlockSpec((BN, 1), lambda i: (i, 0)),
        pl.BlockSpec((BN, 1), lambda i: (i, 0)),
        pl.BlockSpec((1, 1), lambda i: (0, 0)),
    ],
    out_shape=[
        jax.ShapeDtypeStruct((NPAD, D), jnp.float32),
        jax.ShapeDtypeStruct((NPAD, 1), jnp.float32),
        jax.ShapeDtypeStruct((NPAD, 1), jnp.float32),
        jax.ShapeDtypeStruct((1, 1), jnp.float32),
    ],
)


def _tc_final_body(a0_ref, a1_ref, s0_ref, s1_ref, b_ref, out_ref):
    s = s0_ref[0, 0] + s1_ref[0, 0]
    acc = a0_ref[...] + a1_ref[...]
    h = acc / (s + 1e-16)[:, None] + b_ref[...]
    nrm = jnp.sqrt(jnp.sum(h * h, axis=1, keepdims=True))
    out_ref[...] = h / jnp.maximum(nrm, 1e-12)


_tc_final = pl.pallas_call(
    _tc_final_body,
    grid=(GRID,),
    in_specs=[
        pl.BlockSpec((BN, D), lambda i: (i, 0)),
        pl.BlockSpec((BN, D), lambda i: (GRID + i, 0)),
        pl.BlockSpec((1, 1, BN), lambda i: (i, 0, 0)),
        pl.BlockSpec((1, 1, BN), lambda i: (GRID + i, 0, 0)),
        pl.BlockSpec((1, D), lambda i: (0, 0)),
    ],
    out_specs=pl.BlockSpec((BN, D), lambda i: (i, 0)),
    out_shape=jax.ShapeDtypeStruct((NPAD, D), jnp.float32),
)


_sc_mesh = plsc.VectorSubcoreMesh(
    core_axis_name="c", subcore_axis_name="s", num_cores=NC, num_subcores=NS)


@functools.partial(
    pl.kernel,
    mesh=_sc_mesh,
    out_type=[
        jax.ShapeDtypeStruct((NC * NPAD, D), jnp.float32),
        jax.ShapeDtypeStruct((NC * NPAD,), jnp.float32),
    ],
    scratch_types=[
        pltpu.VMEM((NPAD,), jnp.float32),
        pltpu.VMEM((NPAD,), jnp.float32),
        pltpu.VMEM((16,), jnp.float32),
        [pltpu.VMEM((CH,), jnp.int32) for _ in range(3)],
        [pltpu.VMEM((CH,), jnp.int32) for _ in range(3)],
        [pltpu.VMEM((CH + 16,), jnp.float32) for _ in range(3)],
        [pltpu.VMEM((CH, D), jnp.float32) for _ in range(3)],
        pltpu.VMEM_SHARED((NPAD, D), jnp.float32),
        pltpu.VMEM_SHARED((NPAD,), jnp.float32),
        pltpu.VMEM_SHARED((NPAD,), jnp.float32),
        pltpu.VMEM_SHARED((NPAD,), jnp.float32),
        [pltpu.SemaphoreType.DMA for _ in range(3)],
        [pltpu.SemaphoreType.DMA for _ in range(3)],
        [pltpu.SemaphoreType.DMA for _ in range(3)],
    ],
    compiler_params=pltpu.CompilerParams(needs_layout_passes=False),
)
def _sc_edge(src_hbm, dst_hbm, g_hbm, es_hbm, ed_hbm, gmax_hbm,
             acc_out, s_out,
             es_v, ed_v, gmax_v, idx_ss, idx_ds, w_vs, rows_vs,
             acc_sh, s_sh, es_sh, ed_sh, semGs, semSs, semWs):
    cid = lax.axis_index("c")
    sid = lax.axis_index("s")
    r0 = sid * RPT
    zb = rows_vs[0]
    for zr in range(CH):
        for q in range(D // 16):
            zb[zr, pl.ds(q * 16, 16)] = jnp.zeros((16,), jnp.float32)
    for k in range(RPT // CH):
        pltpu.sync_copy(zb, acc_sh.at[pl.ds(r0 + k * CH, CH)])
    for k in range(RPT // 128):
        pltpu.sync_copy(zb.at[0], s_sh.at[pl.ds(r0 + k * 128, 128)])
    @pl.when(sid == 0)
    def _():
        pltpu.sync_copy(es_hbm, es_sh)
        pltpu.sync_copy(ed_hbm, ed_sh)

    pltpu.sync_copy(gmax_hbm, gmax_v)
    plsc.subcore_barrier()
    pltpu.sync_copy(es_sh, es_v)
    pltpu.sync_copy(ed_sh, ed_v)
    plsc.subcore_barrier()

    gmax = gmax_v[...]
    base = (cid * NS + sid) * EPW
    nch = EPW // CH
    bufs = tuple(
        (idx_ss[i], idx_ds[i], w_vs[i], rows_vs[i], semGs[i], semSs[i],
         semWs[i]) for i in range(3))

    def fetch(j, b):
        idx_s, idx_d, _, rows_v, semG, _, _ = b
        off = base + j * CH
        pltpu.sync_copy(src_hbm.at[pl.ds(off, CH)], idx_s)
        pltpu.sync_copy(dst_hbm.at[pl.ds(off, CH)], idx_d)
        pltpu.async_copy(g_hbm.at[idx_s], rows_v, semG)

    def wait_scatter(b):
        _, idx_d, w_v, rows_v, _, semS, semW = b
        pltpu.make_async_copy(rows_v, acc_sh.at[idx_d], semS).wait()
        pltpu.make_async_copy(w_v.at[pl.ds(16, CH)], s_sh.at[idx_d],
                              semW).wait()

    def proc(u, b, b2):
        idx_s, idx_d, w_v, rows_v, semG, semS, semW = b
        pltpu.make_async_copy(g_hbm.at[idx_s], rows_v, semG).wait()
        for k in range(CH // 16):
            sv = idx_s[pl.ds(k * 16, 16)]
            dv = idx_d[pl.ds(k * 16, 16)]
            es16 = plsc.load_gather(es_v, [sv])
            ed16 = plsc.load_gather(ed_v, [dv])
            e = _leaky(es16 + ed16)
            c = _leaky(gmax + ed16)
            w_v[pl.ds(16 + k * 16, 16)] = _exp_neg(e - c)
        for r in range(CH):
            wr = plsc.load_gather(
                w_v, [jnp.full((16,), 16 + r, dtype=jnp.int32)])
            for q in range(D // 16):
                rows_v[r, pl.ds(q * 16, 16)] = rows_v[r, pl.ds(q * 16, 16)] * wr

        @pl.when(u >= 1)
        def _():
            wait_scatter(b2)

        @pl.when(u + 2 < nch)
        def _():
            fetch(u + 2, b2)

        pltpu.async_copy(rows_v, acc_sh.at[idx_d], semS, add=True)
        pltpu.async_copy(w_v.at[pl.ds(16, CH)], s_sh.at[idx_d], semW,
                         add=True)

    fetch(0, bufs[0])
    fetch(1, bufs[1])

    def tri(t, carry):
        u = 3 * t
        proc(u, bufs[0], bufs[2])
        proc(u + 1, bufs[1], bufs[0])
        proc(u + 2, bufs[2], bufs[1])
        return carry

    lax.fori_loop(0, (nch - 1) // 3, tri, 0)
    proc(nch - 1, bufs[0], bufs[2])
    wait_scatter(bufs[0])
    plsc.subcore_barrier()
    out_r0 = cid * NPAD + r0
    pltpu.sync_copy(acc_sh.at[pl.ds(r0, RPT)], acc_out.at[pl.ds(out_r0, RPT)])
    pltpu.sync_copy(s_sh.at[pl.ds(r0, RPT)], s_out.at[pl.ds(out_r0, RPT)])


def kernel(x, edge_index, Wp, bp, Ws, a_src, a_dst, bs):
    xp = jnp.concatenate([x, jnp.zeros((NPAD - N, D), jnp.float32)], axis=0)
    loops = jnp.arange(N, dtype=jnp.int32)
    npad_e = EPAD - (edge_index.shape[1] + N)
    src = jnp.concatenate([
        edge_index[0], loops, jnp.zeros((npad_e,), jnp.int32)])
    dst = jnp.concatenate([
        edge_index[1], loops,
        N + (jnp.arange(npad_e, dtype=jnp.int32) % (NPAD - N))])

    g, es, ed, gmax = _tc_first(
        xp, Wp, bp[None, :], Ws[0], a_src[0][:, None], a_dst[0][:, None])
    for i in range(L):
        gmax16 = jnp.broadcast_to(gmax.reshape(1), (16,))
        acc, s = _sc_edge(src, dst, g, es.reshape(NPAD), ed.reshape(NPAD),
                          gmax16)
        s3 = s.reshape(NC * GRID, 1, BN)
        if i < L - 1:
            g, es, ed, gmax = _tc_mid(
                acc, acc, s3, s3, bs[i][None, :], Ws[i + 1],
                a_src[i + 1][:, None], a_dst[i + 1][:, None])
        else:
            out = _tc_final(acc, acc, s3, s3, bs[i][None, :])
    return out[:N]


# speedup vs baseline: 1.4328x; 1.4264x over previous
import functools

import jax
import jax.numpy as jnp
from jax import lax
from jax.experimental import pallas as pl
from jax.experimental.pallas import tpu as pltpu
from jax.experimental.pallas import tpu_sc as plsc

N = 10000
D = 128
L = 8
NPAD = 10240
NC, NS = 2, 16
NW = NC * NS
CH = 64
EPW = 10432
EPAD = EPW * NW
RPT = NPAD // NS
BN = 320
GRID = NPAD // BN
SLOPE = 0.2


def _leaky(v):
    return jnp.where(v >= 0, v, SLOPE * v)


_LOG2E = 1.4426950408889634
_E2C = [1.0, 0.6931471805599453, 0.2402265069591007, 0.05550410866482158,
        0.009618129107628477, 0.001333355814642844, 0.00015403530393381608]


def _exp_neg(d):
    z = jnp.maximum(d * _LOG2E, -126.0)
    n = (z - 0.5).astype(jnp.int32)
    f = z - n.astype(jnp.float32)
    p = jnp.full((16,), _E2C[6], jnp.float32)
    for c in (_E2C[5], _E2C[4], _E2C[3], _E2C[2], _E2C[1], _E2C[0]):
        p = p * f + c
    scale = jax.lax.bitcast_convert_type(
        jax.lax.shift_left(n + 127, 23), jnp.float32)
    return p * scale


def _acc_gmax(gmax_ref, i, es):
    @pl.when(i == 0)
    def _():
        gmax_ref[...] = jnp.full((1, 1), -jnp.inf, jnp.float32)

    gmax_ref[...] = jnp.maximum(gmax_ref[...], jnp.max(es, keepdims=True))


def _tc_first_body(x_ref, wp_ref, bp_ref, w_ref, asrc_ref, adst_ref,
                   g_ref, es_ref, ed_ref, gmax_ref):
    i = pl.program_id(0)
    h = jnp.dot(x_ref[...], wp_ref[...], preferred_element_type=jnp.float32, precision=jax.lax.Precision.HIGHEST)
    h = h + bp_ref[...]
    g = jnp.dot(h, w_ref[...], preferred_element_type=jnp.float32, precision=jax.lax.Precision.HIGHEST)
    g_ref[...] = g
    es = jnp.dot(g, asrc_ref[...], preferred_element_type=jnp.float32, precision=jax.lax.Precision.HIGHEST)
    ed = jnp.dot(g, adst_ref[...], preferred_element_type=jnp.float32, precision=jax.lax.Precision.HIGHEST)
    es_ref[...] = es
    ed_ref[...] = ed
    _acc_gmax(gmax_ref, i, es)


_tc_first = pl.pallas_call(
    _tc_first_body,
    grid=(GRID,),
    in_specs=[
        pl.BlockSpec((BN, D), lambda i: (i, 0)),
        pl.BlockSpec((D, D), lambda i: (0, 0)),
        pl.BlockSpec((1, D), lambda i: (0, 0)),
        pl.BlockSpec((D, D), lambda i: (0, 0)),
        pl.BlockSpec((D, 1), lambda i: (0, 0)),
        pl.BlockSpec((D, 1), lambda i: (0, 0)),
    ],
    out_specs=[
        pl.BlockSpec((BN, D), lambda i: (i, 0)),
        pl.BlockSpec((BN, 1), lambda i: (i, 0)),
        pl.BlockSpec((BN, 1), lambda i: (i, 0)),
        pl.BlockSpec((1, 1), lambda i: (0, 0)),
    ],
    out_shape=[
        jax.ShapeDtypeStruct((NPAD, D), jnp.float32),
        jax.ShapeDtypeStruct((NPAD, 1), jnp.float32),
        jax.ShapeDtypeStruct((NPAD, 1), jnp.float32),
        jax.ShapeDtypeStruct((1, 1), jnp.float32),
    ],
)


def _tc_mid_body(a0_ref, a1_ref, s0_ref, s1_ref, b_ref, w_ref, asrc_ref,
                 adst_ref, g_ref, es_ref, ed_ref, gmax_ref):
    i = pl.program_id(0)
    s = s0_ref[0, 0] + s1_ref[0, 0]
    acc = a0_ref[...] + a1_ref[...]
    h = acc / (s + 1e-16)[:, None] + b_ref[...]
    h = jnp.maximum(h, 0.0)
    rows = i * BN + lax.broadcasted_iota(jnp.int32, (BN, 1), 0)
    h = jnp.where(rows < N, h, 0.0)
    g = jnp.dot(h, w_ref[...], preferred_element_type=jnp.float32, precision=jax.lax.Precision.HIGHEST)
    g_ref[...] = g
    es = jnp.dot(g, asrc_ref[...], preferred_element_type=jnp.float32, precision=jax.lax.Precision.HIGHEST)
    ed = jnp.dot(g, adst_ref[...], preferred_element_type=jnp.float32, precision=jax.lax.Precision.HIGHEST)
    es_ref[...] = es
    ed_ref[...] = ed
    _acc_gmax(gmax_ref, i, es)


_tc_mid = pl.pallas_call(
    _tc_mid_body,
    grid=(GRID,),
    in_specs=[
        pl.BlockSpec((BN, D), lambda i: (i, 0)),
        pl.BlockSpec((BN, D), lambda i: (GRID + i, 0)),
        pl.BlockSpec((1, 1, BN), lambda i: (i, 0, 0)),
        pl.BlockSpec((1, 1, BN), lambda i: (GRID + i, 0, 0)),
        pl.BlockSpec((1, D), lambda i: (0, 0)),
        pl.BlockSpec((D, D), lambda i: (0, 0)),
        pl.BlockSpec((D, 1), lambda i: (0, 0)),
        pl.BlockSpec((D, 1), lambda i: (0, 0)),
    ],
    out_specs=[
        pl.BlockSpec((BN, D), lambda i: (i, 0)),
        pl.BlockSpec((BN, 1), lambda i: (i, 0)),
        pl.BlockSpec((BN, 1), lambda i: (i, 0)),
        pl.BlockSpec((1, 1), lambda i: (0, 0)),
    ],
    out_shape=[
        jax.ShapeDtypeStruct((NPAD, D), jnp.float32),
        jax.ShapeDtypeStruct((NPAD, 1), jnp.float32),
        jax.ShapeDtypeStruct((NPAD, 1), jnp.float32),
        jax.ShapeDtypeStruct((1, 1), jnp.float32),
    ],
)


def _tc_final_body(a0_ref, a1_ref, s0_ref, s1_ref, b_ref, out_ref):
    s = s0_ref[0, 0] + s1_ref[0, 0]
    acc = a0_ref[...] + a1_ref[...]
    h = acc / (s + 1e-16)[:, None] + b_ref[...]
    nrm = jnp.sqrt(jnp.sum(h * h, axis=1, keepdims=True))
    out_ref[...] = h / jnp.maximum(nrm, 1e-12)


_tc_final = pl.pallas_call(
    _tc_final_body,
    grid=(GRID,),
    in_specs=[
        pl.BlockSpec((BN, D), lambda i: (i, 0)),
        pl.BlockSpec((BN, D), lambda i: (GRID + i, 0)),
        pl.BlockSpec((1, 1, BN), lambda i: (i, 0, 0)),
        pl.BlockSpec((1, 1, BN), lambda i: (GRID + i, 0, 0)),
        pl.BlockSpec((1, D), lambda i: (0, 0)),
    ],
    out_specs=pl.BlockSpec((BN, D), lambda i: (i, 0)),
    out_shape=jax.ShapeDtypeStruct((NPAD, D), jnp.float32),
)


_sc_mesh = plsc.VectorSubcoreMesh(
    core_axis_name="c", subcore_axis_name="s", num_cores=NC, num_subcores=NS)


@functools.partial(
    pl.kernel,
    mesh=_sc_mesh,
    out_type=[
        jax.ShapeDtypeStruct((NC * NPAD, D), jnp.float32),
        jax.ShapeDtypeStruct((NC * NPAD,), jnp.float32),
    ],
    scratch_types=[
        pltpu.VMEM((NPAD,), jnp.float32),
        pltpu.VMEM((NPAD,), jnp.float32),
        pltpu.VMEM((16,), jnp.float32),
        [pltpu.VMEM((CH,), jnp.int32) for _ in range(6)],
        [pltpu.VMEM((CH,), jnp.int32) for _ in range(6)],
        [pltpu.VMEM((CH + 16,), jnp.float32) for _ in range(3)],
        [pltpu.VMEM((CH, D), jnp.float32) for _ in range(3)],
        pltpu.VMEM_SHARED((NPAD, D), jnp.float32),
        pltpu.VMEM_SHARED((NPAD,), jnp.float32),
        pltpu.VMEM_SHARED((NPAD,), jnp.float32),
        pltpu.VMEM_SHARED((NPAD,), jnp.float32),
        [pltpu.SemaphoreType.DMA for _ in range(3)],
        [pltpu.SemaphoreType.DMA for _ in range(3)],
        [pltpu.SemaphoreType.DMA for _ in range(3)],
        [pltpu.SemaphoreType.DMA for _ in range(6)],
    ],
    compiler_params=pltpu.CompilerParams(needs_layout_passes=False),
)
def _sc_edge(src_hbm, dst_hbm, g_hbm, es_hbm, ed_hbm, gmax_hbm,
             acc_out, s_out,
             es_v, ed_v, gmax_v, idx_ss, idx_ds, w_vs, rows_vs,
             acc_sh, s_sh, es_sh, ed_sh, semGs, semSs, semWs, semIs):
    cid = lax.axis_index("c")
    sid = lax.axis_index("s")
    r0 = sid * RPT
    zb = rows_vs[0]
    for zr in range(CH):
        for q in range(D // 16):
            zb[zr, pl.ds(q * 16, 16)] = jnp.zeros((16,), jnp.float32)
    for k in range(RPT // CH):
        pltpu.sync_copy(zb, acc_sh.at[pl.ds(r0 + k * CH, CH)])
    for k in range(RPT // 128):
        pltpu.sync_copy(zb.at[0], s_sh.at[pl.ds(r0 + k * 128, 128)])
    @pl.when(sid == 0)
    def _():
        pltpu.sync_copy(es_hbm, es_sh)
        pltpu.sync_copy(ed_hbm, ed_sh)

    pltpu.sync_copy(gmax_hbm, gmax_v)
    plsc.subcore_barrier()
    pltpu.sync_copy(es_sh, es_v)
    pltpu.sync_copy(ed_sh, ed_v)
    plsc.subcore_barrier()

    gmax = gmax_v[...]
    base = (cid * NS + sid) * EPW
    nch = EPW // CH

    def idx_fetch_sync(j, i):
        off = base + j * CH
        pltpu.sync_copy(src_hbm.at[pl.ds(off, CH)], idx_ss[i])
        pltpu.sync_copy(dst_hbm.at[pl.ds(off, CH)], idx_ds[i])

    def idx_fetch_async(j, i):
        off = base + j * CH
        pltpu.async_copy(src_hbm.at[pl.ds(off, CH)], idx_ss[i], semIs[i])
        pltpu.async_copy(dst_hbm.at[pl.ds(off, CH)], idx_ds[i], semIs[i])

    def idx_wait(j, i):
        off = base + j * CH
        pltpu.make_async_copy(src_hbm.at[pl.ds(off, CH)], idx_ss[i],
                              semIs[i]).wait()
        pltpu.make_async_copy(dst_hbm.at[pl.ds(off, CH)], idx_ds[i],
                              semIs[i]).wait()

    def wait_scatter(B, I):
        pltpu.make_async_copy(rows_vs[B], acc_sh.at[idx_ds[I]],
                              semSs[B]).wait()
        pltpu.make_async_copy(w_vs[B].at[pl.ds(16, CH)], s_sh.at[idx_ds[I]],
                              semWs[B]).wait()

    def proc(u, B, I, b2, I2, I4):
        idx_s, idx_d, w_v, rows_v = idx_ss[I], idx_ds[I], w_vs[B], rows_vs[B]
        pltpu.make_async_copy(g_hbm.at[idx_s], rows_v, semGs[B]).wait()
        for k in range(CH // 16):
            sv = idx_s[pl.ds(k * 16, 16)]
            dv = idx_d[pl.ds(k * 16, 16)]
            es16 = plsc.load_gather(es_v, [sv])
            ed16 = plsc.load_gather(ed_v, [dv])
            e = _leaky(es16 + ed16)
            c = _leaky(gmax + ed16)
            w_v[pl.ds(16 + k * 16, 16)] = _exp_neg(e - c)

        def scale(r, carry):
            wr = plsc.load_gather(
                w_v, [jnp.full((16,), 16, jnp.int32) + r])
            for q in range(D // 16):
                rows_v[r, pl.ds(q * 16, 16)] = rows_v[r, pl.ds(q * 16, 16)] * wr
            return carry

        lax.fori_loop(0, CH, scale, 0)

        @pl.when(u >= 1)
        def _():
            wait_scatter(b2, (I + 5) % 6)

        @pl.when(u + 2 < nch)
        def _():
            idx_wait(u + 2, I2)
            pltpu.async_copy(g_hbm.at[idx_ss[I2]], rows_vs[b2], semGs[b2])

        @pl.when(u + 4 < nch)
        def _():
            idx_fetch_async(u + 4, I4)

        pltpu.async_copy(rows_v, acc_sh.at[idx_d], semSs[B], add=True)
        pltpu.async_copy(w_v.at[pl.ds(16, CH)], s_sh.at[idx_d], semWs[B],
                         add=True)

    idx_fetch_sync(0, 0)
    idx_fetch_sync(1, 1)
    idx_fetch_async(2, 2)
    idx_fetch_async(3, 3)
    pltpu.async_copy(g_hbm.at[idx_ss[0]], rows_vs[0], semGs[0])
    pltpu.async_copy(g_hbm.at[idx_ss[1]], rows_vs[1], semGs[1])

    def hexa(t, carry):
        u = 6 * t
        proc(u, 0, 0, 2, 2, 4)
        proc(u + 1, 1, 1, 0, 3, 5)
        proc(u + 2, 2, 2, 1, 4, 0)
        proc(u + 3, 0, 3, 2, 5, 1)
        proc(u + 4, 1, 4, 0, 0, 2)
        proc(u + 5, 2, 5, 1, 1, 3)
        return carry

    lax.fori_loop(0, (nch - 1) // 6, hexa, 0)
    proc(nch - 1, 0, 0, 2, 2, 4)
    wait_scatter(0, 0)
    plsc.subcore_barrier()
    out_r0 = cid * NPAD + r0
    pltpu.sync_copy(acc_sh.at[pl.ds(r0, RPT)], acc_out.at[pl.ds(out_r0, RPT)])
    pltpu.sync_copy(s_sh.at[pl.ds(r0, RPT)], s_out.at[pl.ds(out_r0, RPT)])


def kernel(x, edge_index, Wp, bp, Ws, a_src, a_dst, bs):
    xp = jnp.concatenate([x, jnp.zeros((NPAD - N, D), jnp.float32)], axis=0)
    loops = jnp.arange(N, dtype=jnp.int32)
    npad_e = EPAD - (edge_index.shape[1] + N)
    src = jnp.concatenate([
        edge_index[0], loops, jnp.zeros((npad_e,), jnp.int32)])
    dst = jnp.concatenate([
        edge_index[1], loops,
        N + (jnp.arange(npad_e, dtype=jnp.int32) % (NPAD - N))])

    g, es, ed, gmax = _tc_first(
        xp, Wp, bp[None, :], Ws[0], a_src[0][:, None], a_dst[0][:, None])
    for i in range(L):
        gmax16 = jnp.broadcast_to(gmax.reshape(1), (16,))
        acc, s = _sc_edge(src, dst, g, es.reshape(NPAD), ed.reshape(NPAD),
                          gmax16)
        s3 = s.reshape(NC * GRID, 1, BN)
        if i < L - 1:
            g, es, ed, gmax = _tc_mid(
                acc, acc, s3, s3, bs[i][None, :], Ws[i + 1],
                a_src[i + 1][:, None], a_dst[i + 1][:, None])
        else:
            out = _tc_final(acc, acc, s3, s3, bs[i][None, :])
    return out[:N]
